# baseline (device time: 774247 ns/iter reference)
import jax
import jax.numpy as jnp
from jax import lax
from jax.experimental import pallas as pl
from jax.experimental.pallas import tpu as pltpu

N_DEV = 16


def kernel(x, w_mat):
    m, k_per = x.shape
    _, n = w_mat.shape
    m_per = m // N_DEV

    def body(x_ref, w_ref, out_ref, comm_ref, send_sems, recv_sems, credit_sem):
        my = lax.axis_index("i")
        left = lax.rem(my + N_DEV - 1, N_DEV)
        right = lax.rem(my + 1, N_DEV)

        barrier_sem = pltpu.get_barrier_semaphore()
        for nbr in (left, right):
            pl.semaphore_signal(barrier_sem, inc=1, device_id=(nbr,),
                                device_id_type=pl.DeviceIdType.MESH)
        pl.semaphore_wait(barrier_sem, 2)

        def partial(c):
            xa = x_ref[pl.ds(c * m_per, m_per), :]
            return jnp.dot(xa, w_ref[:, :], preferred_element_type=jnp.float32)

        c0 = lax.rem(my + N_DEV - 1, N_DEV)
        comm_ref[0, :, :] = partial(c0).astype(jnp.bfloat16)

        acc = None
        for h in range(N_DEV - 1):
            s = h % 2
            r = (h + 1) % 2
            if h >= 2:
                pl.semaphore_wait(credit_sem, 1)
            rdma = pltpu.make_async_remote_copy(
                src_ref=comm_ref.at[s],
                dst_ref=comm_ref.at[r],
                send_sem=send_sems.at[s],
                recv_sem=recv_sems.at[r],
                device_id=(right,),
                device_id_type=pl.DeviceIdType.MESH,
            )
            rdma.start()
            c_recv = lax.rem(my + 2 * N_DEV - 2 - h, N_DEV)
            part = partial(c_recv)
            rdma.wait()
            acc = comm_ref[r, :, :].astype(jnp.float32) + part
            if h < N_DEV - 2:
                comm_ref[r, :, :] = acc.astype(jnp.bfloat16)
            if 1 <= h <= N_DEV - 3:
                pl.semaphore_signal(credit_sem, inc=1, device_id=(left,),
                                    device_id_type=pl.DeviceIdType.MESH)

        y = acc
        c = 0.7978845608028654
        out_ref[:, :] = 0.5 * y * (1.0 + jnp.tanh(c * (y + 0.044715 * y * y * y)))

    return pl.pallas_call(
        body,
        out_shape=jax.ShapeDtypeStruct((m_per, n), jnp.float32),
        in_specs=[
            pl.BlockSpec(memory_space=pltpu.VMEM),
            pl.BlockSpec(memory_space=pltpu.VMEM),
        ],
        out_specs=pl.BlockSpec(memory_space=pltpu.VMEM),
        scratch_shapes=[
            pltpu.VMEM((2, m_per, n), jnp.bfloat16),
            pltpu.SemaphoreType.DMA((2,)),
            pltpu.SemaphoreType.DMA((2,)),
            pltpu.SemaphoreType.REGULAR,
        ],
        compiler_params=pltpu.CompilerParams(collective_id=0),
    )(x, w_mat)


# device time: 439987 ns/iter; 1.7597x vs baseline; 1.7597x over previous
import jax
import jax.numpy as jnp
from jax import lax
from jax.experimental import pallas as pl
from jax.experimental.pallas import tpu as pltpu

N_DEV = 16


def kernel(x, w_mat):
    m, k_per = x.shape
    _, n = w_mat.shape
    m_per = m // N_DEV
    n_half = n // 2

    def body(x_ref, w_ref, out_ref, comm_a, comm_b,
             send_a, recv_a, send_b, recv_b, credit_a, credit_b):
        my = lax.axis_index("i")
        left = lax.rem(my + N_DEV - 1, N_DEV)
        right = lax.rem(my + 1, N_DEV)

        barrier_sem = pltpu.get_barrier_semaphore()
        for nbr in (left, right):
            pl.semaphore_signal(barrier_sem, inc=1, device_id=(nbr,),
                                device_id_type=pl.DeviceIdType.MESH)
        pl.semaphore_wait(barrier_sem, 2)

        def partial_a(c):
            xa = x_ref[pl.ds(c * m_per, m_per), :]
            return jnp.dot(xa, w_ref[:, 0:n_half],
                           preferred_element_type=jnp.float32)

        def partial_b(c):
            xa = x_ref[pl.ds(c * m_per, m_per), :]
            return jnp.dot(xa, w_ref[:, n_half:n],
                           preferred_element_type=jnp.float32)

        comm_a[0, :, :] = partial_a(lax.rem(my + N_DEV - 1, N_DEV)).astype(jnp.bfloat16)
        comm_b[0, :, :] = partial_b(lax.rem(my + 1, N_DEV)).astype(jnp.bfloat16)

        acc_a = None
        acc_b = None
        for h in range(N_DEV - 1):
            s = h % 2
            r = (h + 1) % 2
            if h >= 2:
                pl.semaphore_wait(credit_a, 1)
                pl.semaphore_wait(credit_b, 1)
            rdma_a = pltpu.make_async_remote_copy(
                src_ref=comm_a.at[s],
                dst_ref=comm_a.at[r],
                send_sem=send_a.at[s],
                recv_sem=recv_a.at[r],
                device_id=(right,),
                device_id_type=pl.DeviceIdType.MESH,
            )
            rdma_b = pltpu.make_async_remote_copy(
                src_ref=comm_b.at[s],
                dst_ref=comm_b.at[r],
                send_sem=send_b.at[s],
                recv_sem=recv_b.at[r],
                device_id=(left,),
                device_id_type=pl.DeviceIdType.MESH,
            )
            rdma_a.start()
            rdma_b.start()
            part_a = partial_a(lax.rem(my + 2 * N_DEV - 2 - h, N_DEV))
            part_b = partial_b(lax.rem(my + 2 + h, N_DEV))
            rdma_a.wait()
            rdma_b.wait()
            acc_a = comm_a[r, :, :].astype(jnp.float32) + part_a
            acc_b = comm_b[r, :, :].astype(jnp.float32) + part_b
            if h < N_DEV - 2:
                comm_a[r, :, :] = acc_a.astype(jnp.bfloat16)
                comm_b[r, :, :] = acc_b.astype(jnp.bfloat16)
            if 1 <= h <= N_DEV - 3:
                pl.semaphore_signal(credit_a, inc=1, device_id=(left,),
                                    device_id_type=pl.DeviceIdType.MESH)
                pl.semaphore_signal(credit_b, inc=1, device_id=(right,),
                                    device_id_type=pl.DeviceIdType.MESH)

        c = 0.7978845608028654
        ya, yb = acc_a, acc_b
        out_ref[:, 0:n_half] = 0.5 * ya * (1.0 + jnp.tanh(c * (ya + 0.044715 * ya * ya * ya)))
        out_ref[:, n_half:n] = 0.5 * yb * (1.0 + jnp.tanh(c * (yb + 0.044715 * yb * yb * yb)))

    return pl.pallas_call(
        body,
        out_shape=jax.ShapeDtypeStruct((m_per, n), jnp.float32),
        in_specs=[
            pl.BlockSpec(memory_space=pltpu.VMEM),
            pl.BlockSpec(memory_space=pltpu.VMEM),
        ],
        out_specs=pl.BlockSpec(memory_space=pltpu.VMEM),
        scratch_shapes=[
            pltpu.VMEM((2, m_per, n_half), jnp.bfloat16),
            pltpu.VMEM((2, m_per, n_half), jnp.bfloat16),
            pltpu.SemaphoreType.DMA((2,)),
            pltpu.SemaphoreType.DMA((2,)),
            pltpu.SemaphoreType.DMA((2,)),
            pltpu.SemaphoreType.DMA((2,)),
            pltpu.SemaphoreType.REGULAR,
            pltpu.SemaphoreType.REGULAR,
        ],
        compiler_params=pltpu.CompilerParams(collective_id=0),
    )(x, w_mat)
